# baseline (speedup 1.0000x reference)
import functools

import jax
import jax.numpy as jnp
from jax import lax
from jax.experimental import pallas as pl
from jax.experimental.pallas import tpu as pltpu
from jax.experimental.pallas import tpu_sc as plsc

_D = 128
_B = 16384
_NC = 2
_NS = 16
_NW = _NC * _NS
_BH = _B // 2
_BPW = _BH // _NW
_NCH = 8
_CH = _BPW // _NCH

_mesh = plsc.VectorSubcoreMesh(core_axis_name="c", subcore_axis_name="s")


@functools.partial(
    pl.kernel,
    mesh=_mesh,
    out_type=jax.ShapeDtypeStruct((_BH, _D), jnp.float32),
    scratch_types=[
        pltpu.VMEM_SHARED((_NS, 2, _D), jnp.float32),
        pltpu.VMEM((_BPW,), jnp.int32),
        pltpu.VMEM((_BPW, _D), jnp.float32),
    ] + [pltpu.SemaphoreType.DMA] * 18,
)
def _gather_kernel(table_hbm, idx_hbm, out_hbm, table_s, flags_v, rows_v,
                   sem_t, sem_o, *ksem):
    cid = lax.axis_index("c")
    sid = lax.axis_index("s")
    wid = sid * _NC + cid
    base = wid * _BPW
    isem = list(ksem[:_NCH])
    gsem = list(ksem[_NCH:])
    cp_t = pltpu.async_copy(table_hbm, table_s.at[sid], sem_t)
    icps = [pltpu.async_copy(idx_hbm.at[pl.ds(base + k * _CH, _CH)],
                             flags_v.at[pl.ds(k * _CH, _CH)], isem[k])
            for k in range(_NCH)]
    cp_t.wait()
    gaths = []
    for k in range(_NCH):
        icps[k].wait()
        gaths.append(pltpu.async_copy(
            table_s.at[sid].at[flags_v.at[pl.ds(k * _CH, _CH)]],
            rows_v.at[pl.ds(k * _CH, _CH)], gsem[k]))
    outs = []
    for k in range(_NCH):
        gaths[k].wait()
        outs.append(pltpu.async_copy(
            rows_v.at[pl.ds(k * _CH, _CH)],
            out_hbm.at[pl.ds(base + k * _CH, _CH)], sem_o))
    for o in outs:
        o.wait()


_BLK = 2048
_NB = _BH // _BLK


def _tc_body(flags_ref, table_ref, out_ref):
    f = flags_ref[0, 0, :].astype(jnp.float32)
    pres = table_ref[0, :]
    diff = table_ref[1, :] - pres
    out_ref[...] = pres[None, :] + f[:, None] * diff[None, :]


def kernel(is_absent, present_goal_vector, absent_goal_vector):
    table = jnp.stack([present_goal_vector, absent_goal_vector])
    idx = is_absent.astype(jnp.int32)
    sc_out = _gather_kernel(table, idx)
    tc_out = pl.pallas_call(
        _tc_body,
        grid=(_NB,),
        in_specs=[
            pl.BlockSpec((1, 1, _BLK), lambda i: (i, 0, 0)),
            pl.BlockSpec((2, _D), lambda i: (0, 0)),
        ],
        out_specs=pl.BlockSpec((_BLK, _D), lambda i: (i, 0)),
        out_shape=jax.ShapeDtypeStruct((_BH, _D), jnp.float32),
    )(idx[_BH:].reshape(_NB, 1, _BLK), table)
    return jnp.concatenate([sc_out, tc_out], axis=0)[None]


# speedup vs baseline: 1.3062x; 1.3062x over previous
import functools

import jax
import jax.numpy as jnp
from jax import lax
from jax.experimental import pallas as pl
from jax.experimental.pallas import tpu as pltpu
from jax.experimental.pallas import tpu_sc as plsc

_D = 128
_B = 16384
_NC = 2
_NS = 16
_NW = _NC * _NS
_BPW = _B // _NW
_NCH = 8
_CH = _BPW // _NCH

_mesh = plsc.VectorSubcoreMesh(core_axis_name="c", subcore_axis_name="s")


@functools.partial(
    pl.kernel,
    mesh=_mesh,
    out_type=jax.ShapeDtypeStruct((_B, _D), jnp.float32),
    scratch_types=[
        pltpu.VMEM_SHARED((_NS, 2, _D), jnp.float32),
        pltpu.VMEM((_BPW,), jnp.int32),
        pltpu.VMEM((_BPW, _D), jnp.float32),
    ] + [pltpu.SemaphoreType.DMA] * 18,
)
def _gather_kernel(table_hbm, idx_hbm, out_hbm, table_s, flags_v, rows_v,
                   sem_t, sem_o, *ksem):
    cid = lax.axis_index("c")
    sid = lax.axis_index("s")
    wid = sid * _NC + cid
    base = wid * _BPW
    isem = list(ksem[:_NCH])
    gsem = list(ksem[_NCH:])
    cp_t = pltpu.async_copy(table_hbm, table_s.at[sid], sem_t)
    icps = [pltpu.async_copy(idx_hbm.at[pl.ds(base + k * _CH, _CH)],
                             flags_v.at[pl.ds(k * _CH, _CH)], isem[k])
            for k in range(_NCH)]
    cp_t.wait()
    gaths = []
    for k in range(_NCH):
        icps[k].wait()
        gaths.append(pltpu.async_copy(
            table_s.at[sid].at[flags_v.at[pl.ds(k * _CH, _CH)]],
            rows_v.at[pl.ds(k * _CH, _CH)], gsem[k]))
    outs = []
    for k in range(_NCH):
        gaths[k].wait()
        outs.append(pltpu.async_copy(
            rows_v.at[pl.ds(k * _CH, _CH)],
            out_hbm.at[pl.ds(base + k * _CH, _CH)], sem_o))
    for o in outs:
        o.wait()


def kernel(is_absent, present_goal_vector, absent_goal_vector):
    table = jnp.stack([present_goal_vector, absent_goal_vector])
    idx = is_absent.astype(jnp.int32)
    out = _gather_kernel(table, idx)
    return out[None]
